# baseline (speedup 1.0000x reference)
import functools

import jax
import jax.numpy as jnp
from jax import lax
from jax.experimental import pallas as pl
from jax.experimental.pallas import tpu as pltpu
from jax.experimental.pallas import tpu_sc as plsc

N = 10000
E = 320000
NB = 2048
F = 128

CHUNK = 128
E_PAD = 327680
ROWS = E_PAD // CHUNK
ROWS_PER_SUB = ROWS // 32
STAGE_ROWS = 40
N_DBL = STAGE_ROWS // 2
ACC_R = 10008
STRIPE = 624
HREF = 10008
ZCHUNK = 48
DUMMY_DST = N
FAST_CORE = 1


def _prop_kernel(z_hbm, srcA_hbm, srcB_hbm, dst_hbm, out_hbm,
                 src_st, dst_st, bufs, acc, gsem, ssem):
    c = lax.axis_index("c")
    w = lax.axis_index("s")

    zv = jnp.zeros((16,), jnp.float32)

    def zero_body(t, carry):
        bufs[0, t // 8, pl.ds((t % 8) * 16, 16)] = zv
        return carry

    lax.fori_loop(0, CHUNK * 8, zero_body, 0)
    for t in range(STRIPE // ZCHUNK):
        pltpu.sync_copy(bufs.at[0, pl.ds(0, ZCHUNK)],
                        acc.at[pl.ds(w * STRIPE + t * ZCHUNK, ZCHUNK)])

    @pl.when(w == 0)
    def _():
        pltpu.sync_copy(bufs.at[0, pl.ds(0, ACC_R - 16 * STRIPE)],
                        acc.at[pl.ds(16 * STRIPE, ACC_R - 16 * STRIPE)])

    plsc.subcore_barrier()

    is_fast = c == FAST_CORE
    rps = jnp.where(is_fast, 3 * STAGE_ROWS, STAGE_ROWS)
    core_base = jnp.where(is_fast, 0, 48 * STAGE_ROWS)
    sub_base = core_base + w * rps

    def run_stage(stage_base):
        @pl.when(c == 0)
        def _():
            pltpu.sync_copy(srcA_hbm.at[pl.ds(stage_base, STAGE_ROWS)], src_st)

        @pl.when(c == 1)
        def _():
            pltpu.sync_copy(srcB_hbm.at[pl.ds(stage_base, STAGE_ROWS)], src_st)

        pltpu.sync_copy(dst_hbm.at[pl.ds(stage_base, STAGE_ROWS)], dst_st)

        pltpu.async_copy(z_hbm.at[src_st.at[0]], bufs.at[0], gsem.at[0])

        def dbl_body(g, carry):
            l0 = 2 * g
            l1 = 2 * g + 1
            pltpu.make_async_copy(z_hbm.at[src_st.at[l0]], bufs.at[0],
                                  gsem.at[0]).wait()

            @pl.when(g >= 1)
            def _():
                pltpu.make_async_copy(bufs.at[1], acc.at[dst_st.at[l0]],
                                      ssem.at[1]).wait()

            pltpu.async_copy(z_hbm.at[src_st.at[l1]], bufs.at[1], gsem.at[1])
            pltpu.async_copy(bufs.at[0], acc.at[dst_st.at[l0]], ssem.at[0],
                             add=True)
            pltpu.make_async_copy(z_hbm.at[src_st.at[l1]], bufs.at[1],
                                  gsem.at[1]).wait()

            @pl.when(g < N_DBL - 1)
            def _():
                pltpu.make_async_copy(bufs.at[0], acc.at[dst_st.at[l1]],
                                      ssem.at[0]).wait()
                pltpu.async_copy(z_hbm.at[src_st.at[l1 + 1]], bufs.at[0],
                                 gsem.at[0])

            pltpu.async_copy(bufs.at[1], acc.at[dst_st.at[l1]], ssem.at[1],
                             add=True)
            return carry

        lax.fori_loop(0, N_DBL, dbl_body, 0)
        pltpu.make_async_copy(bufs.at[0], acc.at[dst_st.at[0]],
                              ssem.at[0]).wait()
        pltpu.make_async_copy(bufs.at[1], acc.at[dst_st.at[0]],
                              ssem.at[1]).wait()

    run_stage(sub_base)
    for extra in (1, 2):
        @pl.when(is_fast)
        def _(extra=extra):
            run_stage(sub_base + extra * STAGE_ROWS)

    plsc.subcore_barrier()

    pltpu.sync_copy(acc.at[pl.ds(w * STRIPE, STRIPE)],
                    out_hbm.at[pl.ds(c * HREF + w * STRIPE, STRIPE)])

    @pl.when(w == 0)
    def _():
        pltpu.sync_copy(acc.at[pl.ds(16 * STRIPE, N - 16 * STRIPE)],
                        out_hbm.at[pl.ds(c * HREF + 16 * STRIPE,
                                         N - 16 * STRIPE)])


def _make_prop():
    mesh = plsc.VectorSubcoreMesh(core_axis_name="c", subcore_axis_name="s")
    return functools.partial(
        pl.kernel,
        mesh=mesh,
        out_type=jax.ShapeDtypeStruct((2 * HREF, F), jnp.float32),
        scratch_types=[
            pltpu.VMEM((STAGE_ROWS, CHUNK), jnp.int32),
            pltpu.VMEM((STAGE_ROWS, CHUNK), jnp.int32),
            pltpu.VMEM((2, CHUNK, F), jnp.float32),
            pltpu.VMEM_SHARED((ACC_R, F), jnp.float32),
            pltpu.SemaphoreType.DMA((2,)),
            pltpu.SemaphoreType.DMA((2,)),
        ],
    )(_prop_kernel)


BM = 1000
BK2 = 512
KB = NB // BK2
MB = N // BM


BMC = 1112
MC = HREF // BMC


def _combine_kernel(pa_ref, pb_ref, out_ref):
    out_ref[...] = pa_ref[...] + pb_ref[...]


def _combine_dup(p):
    return pl.pallas_call(
        _combine_kernel,
        grid=(MC, 2),
        in_specs=[pl.BlockSpec((BMC, F), lambda i, h: (i, 0)),
                  pl.BlockSpec((BMC, F), lambda i, h: (i + MC, 0))],
        out_specs=pl.BlockSpec((BMC, F), lambda i, h: (h * MC + i, 0)),
        out_shape=jax.ShapeDtypeStruct((2 * HREF, F), jnp.float32),
    )(p, p)


def _combine_single(p):
    return pl.pallas_call(
        _combine_kernel,
        grid=(MC,),
        in_specs=[pl.BlockSpec((BMC, F), lambda i: (i, 0)),
                  pl.BlockSpec((BMC, F), lambda i: (i + MC, 0))],
        out_specs=pl.BlockSpec((BMC, F), lambda i: (i, 0)),
        out_shape=jax.ShapeDtypeStruct((HREF, F), jnp.float32),
    )(p, p)


def _g_kernel(feat_b_ref, wf_ref, g0_ref, g1_ref):
    fb = feat_b_ref[...]
    g0_ref[...] = lax.dot_general(fb, wf_ref[:, 0:F],
                                  (((1,), (1,)), ((), ())),
                                  preferred_element_type=jnp.float32)
    g1_ref[...] = lax.dot_general(fb, wf_ref[:, F:2 * F],
                                  (((1,), (1,)), ((), ())),
                                  preferred_element_type=jnp.float32)


def _pm_kernel(pm_ref, g0_ref, g1_ref, fa_ref, deg_ref,
               wprev_ref, wdeg_ref, b_ref, out_ref):
    k = pl.program_id(1)

    @pl.when(k == 0)
    def _():
        fa = fa_ref[...]
        d = deg_ref[...]
        dn = (((1,), (1,)), ((), ()))
        small = lax.dot_general(fa, wprev_ref[...], dn,
                                preferred_element_type=jnp.float32)
        small += lax.dot_general(d * fa, wdeg_ref[...], dn,
                                 preferred_element_type=jnp.float32)
        out_ref[...] = small + b_ref[...]

    out_ref[...] += jnp.dot(pm_ref[:, 0, :], g0_ref[...],
                            preferred_element_type=jnp.float32)
    out_ref[...] += jnp.dot(pm_ref[:, 1, :], g1_ref[...],
                            preferred_element_type=jnp.float32)


def _radius_kernel(base_ref, z1_ref, z2_ref, z4_ref,
                   wr0_ref, wr1_ref, wr2_ref, out_ref, stats_ref):
    i = pl.program_id(0)
    dn = (((1,), (1,)), ((), ()))
    res = base_ref[...]
    res += lax.dot_general(z1_ref[...], wr0_ref[...], dn,
                           preferred_element_type=jnp.float32)
    res += lax.dot_general(z2_ref[...], wr1_ref[...], dn,
                           preferred_element_type=jnp.float32)
    res += lax.dot_general(z4_ref[...], wr2_ref[...], dn,
                           preferred_element_type=jnp.float32)
    col = lax.broadcasted_iota(jnp.int32, (BM, F), 1)
    res = jnp.where((col >= F // 2) & (res < 0.0), 0.0, res)
    out_ref[...] = res

    @pl.when(i == 0)
    def _():
        stats_ref[...] = jnp.zeros((8, F), jnp.float32)

    stats_ref[0:1, :] += jnp.sum(res, axis=0, keepdims=True)
    stats_ref[1:2, :] += jnp.sum(res * res, axis=0, keepdims=True)


def _norm_kernel(res_ref, stats_ref, gamma_ref, beta_ref, out_ref):
    mean = stats_ref[0:1, :] / N
    var = stats_ref[1:2, :] / N - mean * mean
    inv = lax.rsqrt(var + 1e-5)
    out_ref[...] = (res_ref[...] - mean) * inv * gamma_ref[...] + beta_ref[...]


def kernel(feat_a, feat_b, deg, pm_pd, edge_index, W_prev, b_prev, W_deg, b_deg,
           W_r0, b_r0, W_r1, b_r1, W_r2, b_r2, W_fuse, b_fuse,
           bn_gamma, bn_beta):
    pad = E_PAD - E
    srcA = jnp.concatenate(
        [edge_index[0], jnp.zeros((pad,), jnp.int32)]).reshape(ROWS, CHUNK)
    srcB = srcA + HREF
    dst_pad = jnp.concatenate(
        [edge_index[1],
         jnp.full((pad,), DUMMY_DST, jnp.int32)]).reshape(ROWS, CHUNK)

    zrow = jnp.zeros((HREF - N, F), jnp.float32)
    feat_dup = jnp.concatenate([feat_a, zrow, feat_a, zrow])

    prop = _make_prop()
    p1 = prop(feat_dup, srcA, srcB, dst_pad)
    z1 = _combine_dup(p1)
    p2 = prop(z1, srcA, srcB, dst_pad)
    z2 = _combine_dup(p2)
    p3 = prop(z2, srcA, srcB, dst_pad)
    z3 = _combine_dup(p3)
    p4 = prop(z3, srcA, srcB, dst_pad)
    z4 = _combine_single(p4)

    g0, g1 = pl.pallas_call(
        _g_kernel,
        out_shape=[jax.ShapeDtypeStruct((NB, F), jnp.float32),
                   jax.ShapeDtypeStruct((NB, F), jnp.float32)],
    )(feat_b, W_fuse)

    pm_t = jnp.transpose(pm_pd, (0, 2, 1))
    b_total = (b_prev + b_deg + b_r0 + b_r1 + b_r2 + b_fuse).reshape(1, F)

    base = pl.pallas_call(
        _pm_kernel,
        grid=(MB, KB),
        in_specs=[
            pl.BlockSpec((BM, 2, BK2), lambda i, k: (i, 0, k)),
            pl.BlockSpec((BK2, F), lambda i, k: (k, 0)),
            pl.BlockSpec((BK2, F), lambda i, k: (k, 0)),
            pl.BlockSpec((BM, F), lambda i, k: (i, 0)),
            pl.BlockSpec((BM, 1), lambda i, k: (i, 0)),
            pl.BlockSpec((F, F), lambda i, k: (0, 0)),
            pl.BlockSpec((F, F), lambda i, k: (0, 0)),
            pl.BlockSpec((1, F), lambda i, k: (0, 0)),
        ],
        out_specs=pl.BlockSpec((BM, F), lambda i, k: (i, 0)),
        out_shape=jax.ShapeDtypeStruct((N, F), jnp.float32),
    )(pm_t, g0, g1, feat_a, deg, W_prev, W_deg, b_total)

    zspec = pl.BlockSpec((BM, F), lambda i: (i, 0))
    wspec = pl.BlockSpec((F, F), lambda i: (0, 0))
    res, stats = pl.pallas_call(
        _radius_kernel,
        grid=(MB,),
        in_specs=[zspec, zspec, zspec, zspec,
                  wspec, wspec, wspec],
        out_specs=[zspec, pl.BlockSpec((8, F), lambda i: (0, 0))],
        out_shape=[jax.ShapeDtypeStruct((N, F), jnp.float32),
                   jax.ShapeDtypeStruct((8, F), jnp.float32)],
    )(base, z1, z2, z4, W_r0, W_r1, W_r2)

    out = pl.pallas_call(
        _norm_kernel,
        grid=(MB,),
        in_specs=[
            zspec,
            pl.BlockSpec((8, F), lambda i: (0, 0)),
            pl.BlockSpec((1, F), lambda i: (0, 0)),
            pl.BlockSpec((1, F), lambda i: (0, 0)),
        ],
        out_specs=zspec,
        out_shape=jax.ShapeDtypeStruct((N, F), jnp.float32),
    )(res, stats, bn_gamma.reshape(1, F), bn_beta.reshape(1, F))

    return out


# speedup vs baseline: 1.0085x; 1.0085x over previous
import functools

import jax
import jax.numpy as jnp
from jax import lax
from jax.experimental import pallas as pl
from jax.experimental.pallas import tpu as pltpu
from jax.experimental.pallas import tpu_sc as plsc

N = 10000
E = 320000
NB = 2048
F = 128

CHUNK = 128
E_PAD = 327680
ROWS = E_PAD // CHUNK
ROWS_PER_SUB = ROWS // 32
STAGE_ROWS = 32
N_DBL = STAGE_ROWS // 2
ACC_R = 10008
STRIPE = 624
HREF = 10008
ZCHUNK = 48
DUMMY_DST = N
FAST_CORE = 1


def _prop_kernel(z_hbm, src_hbm, dst_hbm, out_hbm,
                 src_st, dst_st, bufs, acc, gsem, ssem):
    c = lax.axis_index("c")
    w = lax.axis_index("s")

    zv = jnp.zeros((16,), jnp.float32)

    def zero_body(t, carry):
        bufs[0, t // 8, pl.ds((t % 8) * 16, 16)] = zv
        return carry

    lax.fori_loop(0, CHUNK * 8, zero_body, 0)
    for t in range(STRIPE // ZCHUNK):
        pltpu.sync_copy(bufs.at[0, pl.ds(0, ZCHUNK)],
                        acc.at[pl.ds(w * STRIPE + t * ZCHUNK, ZCHUNK)])

    @pl.when(w == 0)
    def _():
        pltpu.sync_copy(bufs.at[0, pl.ds(0, ACC_R - 16 * STRIPE)],
                        acc.at[pl.ds(16 * STRIPE, ACC_R - 16 * STRIPE)])

    plsc.subcore_barrier()

    is_fast = c == FAST_CORE
    rps = jnp.where(is_fast, 3 * STAGE_ROWS, 2 * STAGE_ROWS)
    core_base = jnp.where(is_fast, 0, 48 * STAGE_ROWS)
    sub_base = core_base + w * rps

    def run_stage(stage_base):
        pltpu.sync_copy(src_hbm.at[pl.ds(stage_base, STAGE_ROWS)], src_st)
        pltpu.sync_copy(dst_hbm.at[pl.ds(stage_base, STAGE_ROWS)], dst_st)

        pltpu.async_copy(z_hbm.at[src_st.at[0]], bufs.at[0], gsem.at[0])

        def dbl_body(g, carry):
            l0 = 2 * g
            l1 = 2 * g + 1
            pltpu.make_async_copy(z_hbm.at[src_st.at[l0]], bufs.at[0],
                                  gsem.at[0]).wait()

            @pl.when(g >= 1)
            def _():
                pltpu.make_async_copy(bufs.at[1], acc.at[dst_st.at[l0]],
                                      ssem.at[1]).wait()

            pltpu.async_copy(z_hbm.at[src_st.at[l1]], bufs.at[1], gsem.at[1])
            pltpu.async_copy(bufs.at[0], acc.at[dst_st.at[l0]], ssem.at[0],
                             add=True)
            pltpu.make_async_copy(z_hbm.at[src_st.at[l1]], bufs.at[1],
                                  gsem.at[1]).wait()

            @pl.when(g < N_DBL - 1)
            def _():
                pltpu.make_async_copy(bufs.at[0], acc.at[dst_st.at[l1]],
                                      ssem.at[0]).wait()
                pltpu.async_copy(z_hbm.at[src_st.at[l1 + 1]], bufs.at[0],
                                 gsem.at[0])

            pltpu.async_copy(bufs.at[1], acc.at[dst_st.at[l1]], ssem.at[1],
                             add=True)
            return carry

        lax.fori_loop(0, N_DBL, dbl_body, 0)
        pltpu.make_async_copy(bufs.at[0], acc.at[dst_st.at[0]],
                              ssem.at[0]).wait()
        pltpu.make_async_copy(bufs.at[1], acc.at[dst_st.at[0]],
                              ssem.at[1]).wait()

    n_stages = jnp.where(is_fast, 3, 2)

    def stage_loop(si, carry):
        run_stage(sub_base + si * STAGE_ROWS)
        return carry

    lax.fori_loop(0, n_stages, stage_loop, 0)

    plsc.subcore_barrier()

    pltpu.sync_copy(acc.at[pl.ds(w * STRIPE, STRIPE)],
                    out_hbm.at[pl.ds(c * HREF + w * STRIPE, STRIPE)])

    @pl.when(w == 0)
    def _():
        pltpu.sync_copy(acc.at[pl.ds(16 * STRIPE, N - 16 * STRIPE)],
                        out_hbm.at[pl.ds(c * HREF + 16 * STRIPE,
                                         N - 16 * STRIPE)])


def _make_prop():
    mesh = plsc.VectorSubcoreMesh(core_axis_name="c", subcore_axis_name="s")
    return functools.partial(
        pl.kernel,
        mesh=mesh,
        out_type=jax.ShapeDtypeStruct((2 * HREF, F), jnp.float32),
        scratch_types=[
            pltpu.VMEM((STAGE_ROWS, CHUNK), jnp.int32),
            pltpu.VMEM((STAGE_ROWS, CHUNK), jnp.int32),
            pltpu.VMEM((2, CHUNK, F), jnp.float32),
            pltpu.VMEM_SHARED((ACC_R, F), jnp.float32),
            pltpu.SemaphoreType.DMA((2,)),
            pltpu.SemaphoreType.DMA((2,)),
        ],
    )(_prop_kernel)


BM = 1000
BK2 = 512
KB = NB // BK2
MB = N // BM


BMC = 1112
MC = HREF // BMC


def _combine_kernel(pa_ref, pb_ref, out_ref):
    out_ref[...] = pa_ref[...] + pb_ref[...]


def _combine_single(p):
    return pl.pallas_call(
        _combine_kernel,
        grid=(MC,),
        in_specs=[pl.BlockSpec((BMC, F), lambda i: (i, 0)),
                  pl.BlockSpec((BMC, F), lambda i: (i + MC, 0))],
        out_specs=pl.BlockSpec((BMC, F), lambda i: (i, 0)),
        out_shape=jax.ShapeDtypeStruct((HREF, F), jnp.float32),
    )(p, p)


def _g_kernel(feat_b_ref, wf_ref, g0_ref, g1_ref):
    fb = feat_b_ref[...]
    g0_ref[...] = lax.dot_general(fb, wf_ref[:, 0:F],
                                  (((1,), (1,)), ((), ())),
                                  preferred_element_type=jnp.float32)
    g1_ref[...] = lax.dot_general(fb, wf_ref[:, F:2 * F],
                                  (((1,), (1,)), ((), ())),
                                  preferred_element_type=jnp.float32)


def _pm_kernel(pm_ref, g0_ref, g1_ref, fa_ref, deg_ref,
               wprev_ref, wdeg_ref, b_ref, out_ref):
    k = pl.program_id(1)

    @pl.when(k == 0)
    def _():
        fa = fa_ref[...]
        d = deg_ref[...]
        dn = (((1,), (1,)), ((), ()))
        small = lax.dot_general(fa, wprev_ref[...], dn,
                                preferred_element_type=jnp.float32)
        small += lax.dot_general(d * fa, wdeg_ref[...], dn,
                                 preferred_element_type=jnp.float32)
        out_ref[...] = small + b_ref[...]

    out_ref[...] += jnp.dot(pm_ref[:, 0, :], g0_ref[...],
                            preferred_element_type=jnp.float32)
    out_ref[...] += jnp.dot(pm_ref[:, 1, :], g1_ref[...],
                            preferred_element_type=jnp.float32)


def _radius_kernel(base_ref, z1_ref, z2_ref, z4_ref,
                   wr0_ref, wr1_ref, wr2_ref, out_ref, stats_ref):
    i = pl.program_id(0)
    dn = (((1,), (1,)), ((), ()))
    res = base_ref[...]
    res += lax.dot_general(z1_ref[...], wr0_ref[...], dn,
                           preferred_element_type=jnp.float32)
    res += lax.dot_general(z2_ref[...], wr1_ref[...], dn,
                           preferred_element_type=jnp.float32)
    res += lax.dot_general(z4_ref[...], wr2_ref[...], dn,
                           preferred_element_type=jnp.float32)
    col = lax.broadcasted_iota(jnp.int32, (BM, F), 1)
    res = jnp.where((col >= F // 2) & (res < 0.0), 0.0, res)
    out_ref[...] = res

    @pl.when(i == 0)
    def _():
        stats_ref[...] = jnp.zeros((8, F), jnp.float32)

    stats_ref[0:1, :] += jnp.sum(res, axis=0, keepdims=True)
    stats_ref[1:2, :] += jnp.sum(res * res, axis=0, keepdims=True)


def _norm_kernel(res_ref, stats_ref, gamma_ref, beta_ref, out_ref):
    mean = stats_ref[0:1, :] / N
    var = stats_ref[1:2, :] / N - mean * mean
    inv = lax.rsqrt(var + 1e-5)
    out_ref[...] = (res_ref[...] - mean) * inv * gamma_ref[...] + beta_ref[...]


def kernel(feat_a, feat_b, deg, pm_pd, edge_index, W_prev, b_prev, W_deg, b_deg,
           W_r0, b_r0, W_r1, b_r1, W_r2, b_r2, W_fuse, b_fuse,
           bn_gamma, bn_beta):
    pad = E_PAD - E
    srcA = jnp.concatenate(
        [edge_index[0], jnp.zeros((pad,), jnp.int32)]).reshape(ROWS, CHUNK)
    dst_pad = jnp.concatenate(
        [edge_index[1],
         jnp.full((pad,), DUMMY_DST, jnp.int32)]).reshape(ROWS, CHUNK)

    feat_p = jnp.concatenate(
        [feat_a, jnp.zeros((HREF - N, F), jnp.float32)])

    prop = _make_prop()
    p1 = prop(feat_p, srcA, dst_pad)
    z1 = _combine_single(p1)
    p2 = prop(z1, srcA, dst_pad)
    z2 = _combine_single(p2)
    p3 = prop(z2, srcA, dst_pad)
    z3 = _combine_single(p3)
    p4 = prop(z3, srcA, dst_pad)
    z4 = _combine_single(p4)

    g0, g1 = pl.pallas_call(
        _g_kernel,
        out_shape=[jax.ShapeDtypeStruct((NB, F), jnp.float32),
                   jax.ShapeDtypeStruct((NB, F), jnp.float32)],
    )(feat_b, W_fuse)

    pm_t = jnp.transpose(pm_pd, (0, 2, 1))
    b_total = (b_prev + b_deg + b_r0 + b_r1 + b_r2 + b_fuse).reshape(1, F)

    base = pl.pallas_call(
        _pm_kernel,
        grid=(MB, KB),
        in_specs=[
            pl.BlockSpec((BM, 2, BK2), lambda i, k: (i, 0, k)),
            pl.BlockSpec((BK2, F), lambda i, k: (k, 0)),
            pl.BlockSpec((BK2, F), lambda i, k: (k, 0)),
            pl.BlockSpec((BM, F), lambda i, k: (i, 0)),
            pl.BlockSpec((BM, 1), lambda i, k: (i, 0)),
            pl.BlockSpec((F, F), lambda i, k: (0, 0)),
            pl.BlockSpec((F, F), lambda i, k: (0, 0)),
            pl.BlockSpec((1, F), lambda i, k: (0, 0)),
        ],
        out_specs=pl.BlockSpec((BM, F), lambda i, k: (i, 0)),
        out_shape=jax.ShapeDtypeStruct((N, F), jnp.float32),
    )(pm_t, g0, g1, feat_a, deg, W_prev, W_deg, b_total)

    zspec = pl.BlockSpec((BM, F), lambda i: (i, 0))
    wspec = pl.BlockSpec((F, F), lambda i: (0, 0))
    res, stats = pl.pallas_call(
        _radius_kernel,
        grid=(MB,),
        in_specs=[zspec, zspec, zspec, zspec,
                  wspec, wspec, wspec],
        out_specs=[zspec, pl.BlockSpec((8, F), lambda i: (0, 0))],
        out_shape=[jax.ShapeDtypeStruct((N, F), jnp.float32),
                   jax.ShapeDtypeStruct((8, F), jnp.float32)],
    )(base, z1, z2, z4, W_r0, W_r1, W_r2)

    out = pl.pallas_call(
        _norm_kernel,
        grid=(MB,),
        in_specs=[
            zspec,
            pl.BlockSpec((8, F), lambda i: (0, 0)),
            pl.BlockSpec((1, F), lambda i: (0, 0)),
            pl.BlockSpec((1, F), lambda i: (0, 0)),
        ],
        out_specs=zspec,
        out_shape=jax.ShapeDtypeStruct((N, F), jnp.float32),
    )(res, stats, bn_gamma.reshape(1, F), bn_beta.reshape(1, F))

    return out


# speedup vs baseline: 1.0596x; 1.0507x over previous
import functools

import jax
import jax.numpy as jnp
from jax import lax
from jax.experimental import pallas as pl
from jax.experimental.pallas import tpu as pltpu
from jax.experimental.pallas import tpu_sc as plsc

N = 10000
E = 320000
NB = 2048
F = 128

CHUNK = 128
E_PAD = 327680
ROWS = E_PAD // CHUNK
ROWS_PER_SUB = ROWS // 32
STAGE_ROWS = 40
N_DBL = STAGE_ROWS // 2
ACC_R = 10008
STRIPE = 624
HREF = 10008
ZCHUNK = 48
DUMMY_DST = N
FAST_CORE = 1


def _prop_kernel(z_hbm, src_hbm, dst_hbm, out_hbm,
                 src_st, dst_st, bufs, acc, gsem, ssem):
    c = lax.axis_index("c")
    w = lax.axis_index("s")

    zv = jnp.zeros((16,), jnp.float32)

    def zero_body(t, carry):
        bufs[0, t // 8, pl.ds((t % 8) * 16, 16)] = zv
        return carry

    lax.fori_loop(0, CHUNK * 8, zero_body, 0)
    for t in range(STRIPE // ZCHUNK):
        pltpu.sync_copy(bufs.at[0, pl.ds(0, ZCHUNK)],
                        acc.at[pl.ds(w * STRIPE + t * ZCHUNK, ZCHUNK)])

    @pl.when(w == 0)
    def _():
        pltpu.sync_copy(bufs.at[0, pl.ds(0, ACC_R - 16 * STRIPE)],
                        acc.at[pl.ds(16 * STRIPE, ACC_R - 16 * STRIPE)])

    plsc.subcore_barrier()

    is_fast = c == FAST_CORE
    rps = jnp.where(is_fast, 3 * STAGE_ROWS, 1 * STAGE_ROWS)
    core_base = jnp.where(is_fast, 0, 48 * STAGE_ROWS)
    sub_base = core_base + w * rps

    def run_stage(stage_base):
        pltpu.sync_copy(src_hbm.at[pl.ds(stage_base, STAGE_ROWS)], src_st)
        pltpu.sync_copy(dst_hbm.at[pl.ds(stage_base, STAGE_ROWS)], dst_st)

        pltpu.async_copy(z_hbm.at[src_st.at[0]], bufs.at[0], gsem.at[0])

        def dbl_body(g, carry):
            l0 = 2 * g
            l1 = 2 * g + 1
            pltpu.make_async_copy(z_hbm.at[src_st.at[l0]], bufs.at[0],
                                  gsem.at[0]).wait()

            @pl.when(g >= 1)
            def _():
                pltpu.make_async_copy(bufs.at[1], acc.at[dst_st.at[l0]],
                                      ssem.at[1]).wait()

            pltpu.async_copy(z_hbm.at[src_st.at[l1]], bufs.at[1], gsem.at[1])
            pltpu.async_copy(bufs.at[0], acc.at[dst_st.at[l0]], ssem.at[0],
                             add=True)
            pltpu.make_async_copy(z_hbm.at[src_st.at[l1]], bufs.at[1],
                                  gsem.at[1]).wait()

            @pl.when(g < N_DBL - 1)
            def _():
                pltpu.make_async_copy(bufs.at[0], acc.at[dst_st.at[l1]],
                                      ssem.at[0]).wait()
                pltpu.async_copy(z_hbm.at[src_st.at[l1 + 1]], bufs.at[0],
                                 gsem.at[0])

            pltpu.async_copy(bufs.at[1], acc.at[dst_st.at[l1]], ssem.at[1],
                             add=True)
            return carry

        lax.fori_loop(0, N_DBL, dbl_body, 0)
        pltpu.make_async_copy(bufs.at[0], acc.at[dst_st.at[0]],
                              ssem.at[0]).wait()
        pltpu.make_async_copy(bufs.at[1], acc.at[dst_st.at[0]],
                              ssem.at[1]).wait()

    n_stages = jnp.where(is_fast, 3, 1)

    def stage_loop(si, carry):
        run_stage(sub_base + si * STAGE_ROWS)
        return carry

    lax.fori_loop(0, n_stages, stage_loop, 0)

    plsc.subcore_barrier()

    pltpu.sync_copy(acc.at[pl.ds(w * STRIPE, STRIPE)],
                    out_hbm.at[pl.ds(c * HREF + w * STRIPE, STRIPE)])

    @pl.when(w == 0)
    def _():
        pltpu.sync_copy(acc.at[pl.ds(16 * STRIPE, N - 16 * STRIPE)],
                        out_hbm.at[pl.ds(c * HREF + 16 * STRIPE,
                                         N - 16 * STRIPE)])


def _make_prop():
    mesh = plsc.VectorSubcoreMesh(core_axis_name="c", subcore_axis_name="s")
    return functools.partial(
        pl.kernel,
        mesh=mesh,
        out_type=jax.ShapeDtypeStruct((2 * HREF, F), jnp.float32),
        scratch_types=[
            pltpu.VMEM((STAGE_ROWS, CHUNK), jnp.int32),
            pltpu.VMEM((STAGE_ROWS, CHUNK), jnp.int32),
            pltpu.VMEM((2, CHUNK, F), jnp.float32),
            pltpu.VMEM_SHARED((ACC_R, F), jnp.float32),
            pltpu.SemaphoreType.DMA((2,)),
            pltpu.SemaphoreType.DMA((2,)),
        ],
    )(_prop_kernel)


BM = 1000
BK2 = 512
KB = NB // BK2
MB = N // BM


BMC = 1112
MC = HREF // BMC


def _combine_kernel(pa_ref, pb_ref, out_ref):
    out_ref[...] = pa_ref[...] + pb_ref[...]


def _combine_single(p):
    return pl.pallas_call(
        _combine_kernel,
        grid=(MC,),
        in_specs=[pl.BlockSpec((BMC, F), lambda i: (i, 0)),
                  pl.BlockSpec((BMC, F), lambda i: (i + MC, 0))],
        out_specs=pl.BlockSpec((BMC, F), lambda i: (i, 0)),
        out_shape=jax.ShapeDtypeStruct((HREF, F), jnp.float32),
    )(p, p)


def _g_kernel(feat_b_ref, wf_ref, g0_ref, g1_ref):
    fb = feat_b_ref[...]
    g0_ref[...] = lax.dot_general(fb, wf_ref[:, 0:F],
                                  (((1,), (1,)), ((), ())),
                                  preferred_element_type=jnp.float32)
    g1_ref[...] = lax.dot_general(fb, wf_ref[:, F:2 * F],
                                  (((1,), (1,)), ((), ())),
                                  preferred_element_type=jnp.float32)


def _pm_kernel(pm_ref, g0_ref, g1_ref, fa_ref, deg_ref,
               wprev_ref, wdeg_ref, b_ref, out_ref):
    k = pl.program_id(1)

    @pl.when(k == 0)
    def _():
        fa = fa_ref[...]
        d = deg_ref[...]
        dn = (((1,), (1,)), ((), ()))
        small = lax.dot_general(fa, wprev_ref[...], dn,
                                preferred_element_type=jnp.float32)
        small += lax.dot_general(d * fa, wdeg_ref[...], dn,
                                 preferred_element_type=jnp.float32)
        out_ref[...] = small + b_ref[...]

    out_ref[...] += jnp.dot(pm_ref[:, 0, :], g0_ref[...],
                            preferred_element_type=jnp.float32)
    out_ref[...] += jnp.dot(pm_ref[:, 1, :], g1_ref[...],
                            preferred_element_type=jnp.float32)


def _radius_kernel(base_ref, z1_ref, z2_ref, z4_ref,
                   wr0_ref, wr1_ref, wr2_ref, out_ref, stats_ref):
    i = pl.program_id(0)
    dn = (((1,), (1,)), ((), ()))
    res = base_ref[...]
    res += lax.dot_general(z1_ref[...], wr0_ref[...], dn,
                           preferred_element_type=jnp.float32)
    res += lax.dot_general(z2_ref[...], wr1_ref[...], dn,
                           preferred_element_type=jnp.float32)
    res += lax.dot_general(z4_ref[...], wr2_ref[...], dn,
                           preferred_element_type=jnp.float32)
    col = lax.broadcasted_iota(jnp.int32, (BM, F), 1)
    res = jnp.where((col >= F // 2) & (res < 0.0), 0.0, res)
    out_ref[...] = res

    @pl.when(i == 0)
    def _():
        stats_ref[...] = jnp.zeros((8, F), jnp.float32)

    stats_ref[0:1, :] += jnp.sum(res, axis=0, keepdims=True)
    stats_ref[1:2, :] += jnp.sum(res * res, axis=0, keepdims=True)


def _norm_kernel(res_ref, stats_ref, gamma_ref, beta_ref, out_ref):
    mean = stats_ref[0:1, :] / N
    var = stats_ref[1:2, :] / N - mean * mean
    inv = lax.rsqrt(var + 1e-5)
    out_ref[...] = (res_ref[...] - mean) * inv * gamma_ref[...] + beta_ref[...]


def kernel(feat_a, feat_b, deg, pm_pd, edge_index, W_prev, b_prev, W_deg, b_deg,
           W_r0, b_r0, W_r1, b_r1, W_r2, b_r2, W_fuse, b_fuse,
           bn_gamma, bn_beta):
    pad = E_PAD - E
    srcA = jnp.concatenate(
        [edge_index[0], jnp.zeros((pad,), jnp.int32)]).reshape(ROWS, CHUNK)
    dst_pad = jnp.concatenate(
        [edge_index[1],
         jnp.full((pad,), DUMMY_DST, jnp.int32)]).reshape(ROWS, CHUNK)

    feat_p = jnp.concatenate(
        [feat_a, jnp.zeros((HREF - N, F), jnp.float32)])

    prop = _make_prop()
    p1 = prop(feat_p, srcA, dst_pad)
    z1 = _combine_single(p1)
    p2 = prop(z1, srcA, dst_pad)
    z2 = _combine_single(p2)
    p3 = prop(z2, srcA, dst_pad)
    z3 = _combine_single(p3)
    p4 = prop(z3, srcA, dst_pad)
    z4 = _combine_single(p4)

    g0, g1 = pl.pallas_call(
        _g_kernel,
        out_shape=[jax.ShapeDtypeStruct((NB, F), jnp.float32),
                   jax.ShapeDtypeStruct((NB, F), jnp.float32)],
    )(feat_b, W_fuse)

    pm_t = jnp.transpose(pm_pd, (0, 2, 1))
    b_total = (b_prev + b_deg + b_r0 + b_r1 + b_r2 + b_fuse).reshape(1, F)

    base = pl.pallas_call(
        _pm_kernel,
        grid=(MB, KB),
        in_specs=[
            pl.BlockSpec((BM, 2, BK2), lambda i, k: (i, 0, k)),
            pl.BlockSpec((BK2, F), lambda i, k: (k, 0)),
            pl.BlockSpec((BK2, F), lambda i, k: (k, 0)),
            pl.BlockSpec((BM, F), lambda i, k: (i, 0)),
            pl.BlockSpec((BM, 1), lambda i, k: (i, 0)),
            pl.BlockSpec((F, F), lambda i, k: (0, 0)),
            pl.BlockSpec((F, F), lambda i, k: (0, 0)),
            pl.BlockSpec((1, F), lambda i, k: (0, 0)),
        ],
        out_specs=pl.BlockSpec((BM, F), lambda i, k: (i, 0)),
        out_shape=jax.ShapeDtypeStruct((N, F), jnp.float32),
    )(pm_t, g0, g1, feat_a, deg, W_prev, W_deg, b_total)

    zspec = pl.BlockSpec((BM, F), lambda i: (i, 0))
    wspec = pl.BlockSpec((F, F), lambda i: (0, 0))
    res, stats = pl.pallas_call(
        _radius_kernel,
        grid=(MB,),
        in_specs=[zspec, zspec, zspec, zspec,
                  wspec, wspec, wspec],
        out_specs=[zspec, pl.BlockSpec((8, F), lambda i: (0, 0))],
        out_shape=[jax.ShapeDtypeStruct((N, F), jnp.float32),
                   jax.ShapeDtypeStruct((8, F), jnp.float32)],
    )(base, z1, z2, z4, W_r0, W_r1, W_r2)

    out = pl.pallas_call(
        _norm_kernel,
        grid=(MB,),
        in_specs=[
            zspec,
            pl.BlockSpec((8, F), lambda i: (0, 0)),
            pl.BlockSpec((1, F), lambda i: (0, 0)),
            pl.BlockSpec((1, F), lambda i: (0, 0)),
        ],
        out_specs=zspec,
        out_shape=jax.ShapeDtypeStruct((N, F), jnp.float32),
    )(res, stats, bn_gamma.reshape(1, F), bn_beta.reshape(1, F))

    return out
